# baseline (speedup 1.0000x reference)
import functools

import jax
import jax.numpy as jnp
from jax import lax
from jax.experimental import pallas as pl
from jax.experimental.pallas import tpu as pltpu
from jax.experimental.pallas import tpu_sc as plsc

N = 50000
E = 1600000
F = 16
OUT = 128
NC, NS = 2, 16
NW = NC * NS
EW = E // NW
B = 2000
NB = EW // B
SB = 80
K = B // SB
NP = 50048
ZR = NP // NS


def _sc_segment_sum(tc2, sc1, dst3):
    mesh = plsc.VectorSubcoreMesh(core_axis_name="c", subcore_axis_name="s")

    @functools.partial(
        pl.kernel,
        out_type=jax.ShapeDtypeStruct((NC, NP, F), jnp.float32),
        mesh=mesh,
        compiler_params=pltpu.CompilerParams(use_tc_tiling_on_sc=False),
        scratch_types=[
            pltpu.VMEM((B * F,), jnp.float32),
            pltpu.VMEM((B, F), jnp.float32),
            pltpu.VMEM((B,), jnp.float32),
            pltpu.VMEM((B,), jnp.int32),
            pltpu.VMEM_SHARED((NP, F), jnp.float32),
        ],
    )
    def k(tc_hbm, sc_hbm, dst_hbm, out_hbm, tcf_buf, tc_buf, sc_buf, dst_buf,
          acc):
        c = lax.axis_index("c")
        s = lax.axis_index("s")
        wid = s * NC + c

        @pl.loop(0, B)
        def _z(i):
            tc_buf[i, :] = jnp.zeros((F,), jnp.float32)

        pltpu.sync_copy(tc_buf, acc.at[pl.ds(s * ZR, B), :])
        pltpu.sync_copy(tc_buf.at[pl.ds(0, ZR - B), :],
                        acc.at[pl.ds(s * ZR + B, ZR - B), :])
        plsc.subcore_barrier()

        @pl.loop(0, NB)
        def _blk(blk):
            base = wid * EW + blk * B
            pltpu.sync_copy(tc_hbm.at[pl.ds(base * F, B * F)], tcf_buf)
            pltpu.sync_copy(sc_hbm.at[pl.ds(base, B)], sc_buf)
            pltpu.sync_copy(dst_hbm.at[pl.ds(base, B)], dst_buf)

            @plsc.parallel_loop(0, B // 16, 1, unroll=2)
            def _mul(gi):
                g = sc_buf[pl.ds(gi * 16, 16)]
                for r in range(16):
                    i = gi * 16 + r
                    tc_buf[i, :] = tcf_buf[pl.ds(i * F, F)] * g[r]

            for j in range(K):
                pltpu.sync_copy(tc_buf.at[pl.ds(j * SB, SB), :],
                                acc.at[dst_buf.at[pl.ds(j * SB, SB)]], add=True)

        plsc.subcore_barrier()

        pltpu.sync_copy(acc.at[pl.ds(s * ZR, B), :], tc_buf)
        pltpu.sync_copy(tc_buf, out_hbm.at[c, pl.ds(s * ZR, B), :])
        pltpu.sync_copy(acc.at[pl.ds(s * ZR + B, ZR - B), :],
                        tc_buf.at[pl.ds(0, ZR - B), :])
        pltpu.sync_copy(tc_buf.at[pl.ds(0, ZR - B), :],
                        out_hbm.at[c, pl.ds(s * ZR + B, ZR - B), :])

    return k(tc2, sc1, dst3)


def _tc_linear(parts, W, b2):
    R = 400
    G = N // R

    def mm(p_ref, w_ref, b_ref, o_ref):
        a = p_ref[0] + p_ref[1]
        acc = lax.dot_general(a, w_ref[...], (((1,), (1,)), ((), ())),
                              preferred_element_type=jnp.float32)
        o_ref[...] = acc + b_ref[...]

    return pl.pallas_call(
        mm,
        grid=(G,),
        in_specs=[
            pl.BlockSpec((NC, R, F), lambda i: (0, i, 0)),
            pl.BlockSpec((OUT, F), lambda i: (0, 0)),
            pl.BlockSpec((1, OUT), lambda i: (0, 0)),
        ],
        out_specs=pl.BlockSpec((R, OUT), lambda i: (i, 0)),
        out_shape=jax.ShapeDtypeStruct((N, OUT), jnp.float32),
    )(parts, W, b2)


def kernel(feature, edge_tc, edge_sc, W, b, edge_index):
    del feature
    parts = _sc_segment_sum(edge_tc.reshape(E * F), edge_sc.reshape(E),
                            edge_index[1])
    return _tc_linear(parts, W, b.reshape(1, OUT))


# speedup vs baseline: 1.0645x; 1.0645x over previous
import functools

import jax
import jax.numpy as jnp
from jax import lax
from jax.experimental import pallas as pl
from jax.experimental.pallas import tpu as pltpu
from jax.experimental.pallas import tpu_sc as plsc

N = 50000
E = 1600000
F = 16
OUT = 128
NC, NS = 2, 16
NW = NC * NS
EW = E // NW
B = 2000
NB = EW // B
SB = 80
K = B // SB
NP = 50048
ZR = NP // NS


def _sc_segment_sum(tc2, sc1, dst3):
    mesh = plsc.VectorSubcoreMesh(core_axis_name="c", subcore_axis_name="s")

    @functools.partial(
        pl.kernel,
        out_type=jax.ShapeDtypeStruct((NC, NP, F), jnp.float32),
        mesh=mesh,
        compiler_params=pltpu.CompilerParams(use_tc_tiling_on_sc=False),
        scratch_types=[
            pltpu.VMEM((B, F), jnp.float32),
            pltpu.VMEM((B,), jnp.float32),
            pltpu.VMEM((B,), jnp.int32),
            pltpu.VMEM_SHARED((NP, F), jnp.float32),
        ],
    )
    def k(tc_hbm, sc_hbm, dst_hbm, out_hbm, tc_buf, sc_buf, dst_buf, acc):
        c = lax.axis_index("c")
        s = lax.axis_index("s")
        wid = s * NC + c

        @pl.loop(0, B)
        def _z(i):
            tc_buf[i, :] = jnp.zeros((F,), jnp.float32)

        pltpu.sync_copy(tc_buf, acc.at[pl.ds(s * ZR, B), :])
        pltpu.sync_copy(tc_buf.at[pl.ds(0, ZR - B), :],
                        acc.at[pl.ds(s * ZR + B, ZR - B), :])
        plsc.subcore_barrier()

        @pl.loop(0, NB)
        def _blk(blk):
            base = wid * EW + blk * B
            j = wid * NB + blk
            pltpu.sync_copy(
                tc_hbm.at[pl.ds((j // 8) * B, B), pl.ds((j % 8) * F, F)],
                tc_buf)
            pltpu.sync_copy(sc_hbm.at[pl.ds(base, B)], sc_buf)
            pltpu.sync_copy(dst_hbm.at[pl.ds(base, B)], dst_buf)

            @plsc.parallel_loop(0, B // 16, 1, unroll=2)
            def _mul(gi):
                g = sc_buf[pl.ds(gi * 16, 16)]
                for r in range(16):
                    tc_buf[gi * 16 + r, :] = tc_buf[gi * 16 + r, :] * g[r]

            for j in range(K):
                pltpu.sync_copy(tc_buf.at[pl.ds(j * SB, SB), :],
                                acc.at[dst_buf.at[pl.ds(j * SB, SB)]], add=True)

        plsc.subcore_barrier()

        pltpu.sync_copy(acc.at[pl.ds(s * ZR, B), :], tc_buf)
        pltpu.sync_copy(tc_buf, out_hbm.at[c, pl.ds(s * ZR, B), :])
        pltpu.sync_copy(acc.at[pl.ds(s * ZR + B, ZR - B), :],
                        tc_buf.at[pl.ds(0, ZR - B), :])
        pltpu.sync_copy(tc_buf.at[pl.ds(0, ZR - B), :],
                        out_hbm.at[c, pl.ds(s * ZR + B, ZR - B), :])

    return k(tc2, sc1, dst3)


def _tc_transpose(tcT):
    Bt = 8 * B
    G = E // Bt

    def tr(x_ref, o_ref):
        ey = jnp.eye(F, dtype=jnp.float32)
        z = jax.lax.dot_general(x_ref[...], ey, (((0,), (0,)), ((), ())),
                                preferred_element_type=jnp.float32)
        o_ref[...] = jnp.concatenate(
            [z[g * B:(g + 1) * B, :] for g in range(8)], axis=1)

    return pl.pallas_call(
        tr,
        grid=(G,),
        in_specs=[pl.BlockSpec((F, Bt), lambda i: (0, i))],
        out_specs=pl.BlockSpec((B, 8 * F), lambda i: (i, 0)),
        out_shape=jax.ShapeDtypeStruct((E // 8, 8 * F), jnp.float32),
    )(tcT)


def _tc_linear(parts, W, b2):
    R = 400
    G = N // R

    def mm(p_ref, w_ref, b_ref, o_ref):
        a = p_ref[0] + p_ref[1]
        acc = lax.dot_general(a, w_ref[...], (((1,), (1,)), ((), ())),
                              preferred_element_type=jnp.float32)
        o_ref[...] = acc + b_ref[...]

    return pl.pallas_call(
        mm,
        grid=(G,),
        in_specs=[
            pl.BlockSpec((NC, R, F), lambda i: (0, i, 0)),
            pl.BlockSpec((OUT, F), lambda i: (0, 0)),
            pl.BlockSpec((1, OUT), lambda i: (0, 0)),
        ],
        out_specs=pl.BlockSpec((R, OUT), lambda i: (i, 0)),
        out_shape=jax.ShapeDtypeStruct((N, OUT), jnp.float32),
    )(parts, W, b2)


def kernel(feature, edge_tc, edge_sc, W, b, edge_index):
    del feature
    m2 = _tc_transpose(edge_tc.T)
    parts = _sc_segment_sum(m2, edge_sc.reshape(E), edge_index[1])
    return _tc_linear(parts, W, b.reshape(1, OUT))


# speedup vs baseline: 1.3140x; 1.2344x over previous
import functools

import jax
import jax.numpy as jnp
from jax import lax
from jax.experimental import pallas as pl
from jax.experimental.pallas import tpu as pltpu
from jax.experimental.pallas import tpu_sc as plsc

N = 50000
E = 1600000
F = 16
OUT = 128
NC, NS = 2, 16
NW = NC * NS
EW = E // NW
B = 2000
NB = EW // B
SB = 80
K = B // SB
NP = 50048
ZR = NP // NS
BP = 2048


def _sc_segment_sum(tc2, sc1, dst3):
    mesh = plsc.VectorSubcoreMesh(core_axis_name="c", subcore_axis_name="s")

    @functools.partial(
        pl.kernel,
        out_type=jax.ShapeDtypeStruct((NC, NP, F), jnp.float32),
        mesh=mesh,
        compiler_params=pltpu.CompilerParams(use_tc_tiling_on_sc=False),
        scratch_types=[
            pltpu.VMEM((B, F), jnp.float32),
            pltpu.VMEM((1, B), jnp.float32),
            pltpu.VMEM((1, B), jnp.int32),
            pltpu.VMEM_SHARED((NP, F), jnp.float32),
        ],
    )
    def k(tc_hbm, sc_hbm, dst_hbm, out_hbm, tc_buf, sc_buf, dst_buf, acc):
        c = lax.axis_index("c")
        s = lax.axis_index("s")
        wid = s * NC + c

        @pl.loop(0, B)
        def _z(i):
            tc_buf[i, :] = jnp.zeros((F,), jnp.float32)

        pltpu.sync_copy(tc_buf, acc.at[pl.ds(s * ZR, B), :])
        pltpu.sync_copy(tc_buf.at[pl.ds(0, ZR - B), :],
                        acc.at[pl.ds(s * ZR + B, ZR - B), :])
        plsc.subcore_barrier()

        @pl.loop(0, NB)
        def _blk(blk):
            base = wid * EW + blk * B
            j = wid * NB + blk
            pltpu.sync_copy(
                tc_hbm.at[pl.ds((j // 8) * B, B), pl.ds((j % 8) * F, F)],
                tc_buf)
            pltpu.sync_copy(sc_hbm.at[pl.ds(0, 1), pl.ds(base, B)], sc_buf)
            pltpu.sync_copy(dst_hbm.at[pl.ds(1, 1), pl.ds(base, B)], dst_buf)

            @plsc.parallel_loop(0, B // 16, 1, unroll=2)
            def _mul(gi):
                g = sc_buf[0, pl.ds(gi * 16, 16)]
                for r in range(16):
                    tc_buf[gi * 16 + r, :] = tc_buf[gi * 16 + r, :] * g[r]

            for q in range(K):
                pltpu.sync_copy(tc_buf.at[pl.ds(q * SB, SB), :],
                                acc.at[dst_buf.at[0, pl.ds(q * SB, SB)]],
                                add=True)

        plsc.subcore_barrier()

        pltpu.sync_copy(acc.at[pl.ds(s * ZR, B), :], tc_buf)
        pltpu.sync_copy(tc_buf, out_hbm.at[c, pl.ds(s * ZR, B), :])
        pltpu.sync_copy(acc.at[pl.ds(s * ZR + B, ZR - B), :],
                        tc_buf.at[pl.ds(0, ZR - B), :])
        pltpu.sync_copy(tc_buf.at[pl.ds(0, ZR - B), :],
                        out_hbm.at[c, pl.ds(s * ZR + B, ZR - B), :])

    return k(tc2, sc1, dst3)


def _tc_transpose(tcT):
    Bt = 8 * B
    G = E // Bt

    def tr(x_ref, o_ref):
        row = jax.lax.broadcasted_iota(jnp.int32, (F, 8 * F), 0)
        col = jax.lax.broadcasted_iota(jnp.int32, (F, 8 * F), 1)
        acc = jnp.zeros((B, 8 * F), jnp.float32)
        for g in range(8):
            sel_g = jnp.where(col == row + g * F, 1.0, 0.0).astype(jnp.float32)
            acc = acc + jax.lax.dot_general(
                x_ref[:, g * B:(g + 1) * B], sel_g, (((0,), (0,)), ((), ())),
                preferred_element_type=jnp.float32)
        o_ref[...] = acc

    return pl.pallas_call(
        tr,
        grid=(G,),
        in_specs=[pl.BlockSpec((F, Bt), lambda i: (0, i))],
        out_specs=pl.BlockSpec((B, 8 * F), lambda i: (i, 0)),
        out_shape=jax.ShapeDtypeStruct((E // 8, 8 * F), jnp.float32),
        compiler_params=pltpu.CompilerParams(fuse_transposed_lhs_in_matmul=True),
    )(tcT)


def _tc_linear(parts, W, b2):
    R = 400
    G = N // R

    def mm(p_ref, w_ref, b_ref, o_ref):
        a = p_ref[0] + p_ref[1]
        acc = lax.dot_general(a, w_ref[...], (((1,), (1,)), ((), ())),
                              preferred_element_type=jnp.float32)
        o_ref[...] = acc + b_ref[...]

    return pl.pallas_call(
        mm,
        grid=(G,),
        in_specs=[
            pl.BlockSpec((NC, R, F), lambda i: (0, i, 0)),
            pl.BlockSpec((OUT, F), lambda i: (0, 0)),
            pl.BlockSpec((1, OUT), lambda i: (0, 0)),
        ],
        out_specs=pl.BlockSpec((R, OUT), lambda i: (i, 0)),
        out_shape=jax.ShapeDtypeStruct((N, OUT), jnp.float32),
    )(parts, W, b2)


def kernel(feature, edge_tc, edge_sc, W, b, edge_index):
    del feature
    m2 = _tc_transpose(edge_tc.T)
    parts = _sc_segment_sum(m2, edge_sc.T, edge_index)
    return _tc_linear(parts, W, b.reshape(1, OUT))


# speedup vs baseline: 1.4894x; 1.1335x over previous
import functools

import jax
import jax.numpy as jnp
from jax import lax
from jax.experimental import pallas as pl
from jax.experimental.pallas import tpu as pltpu
from jax.experimental.pallas import tpu_sc as plsc

N = 50000
E = 1600000
F = 16
OUT = 128
NC, NS = 2, 16
NW = NC * NS
EW = E // NW
B = 2000
NB = EW // B
SB = 80
K = B // SB
NP = 50048
ZR = NP // NS
BP = 2048


def _sc_segment_sum(tc2, sc1, dst3):
    mesh = plsc.VectorSubcoreMesh(core_axis_name="c", subcore_axis_name="s")

    @functools.partial(
        pl.kernel,
        out_type=jax.ShapeDtypeStruct((NC, NP, F), jnp.float32),
        mesh=mesh,
        compiler_params=pltpu.CompilerParams(use_tc_tiling_on_sc=False),
        scratch_types=[
            pltpu.VMEM((B, F), jnp.float32),
            pltpu.VMEM((1, B), jnp.float32),
            pltpu.VMEM((1, B), jnp.int32),
            pltpu.VMEM_SHARED((NP, F), jnp.float32),
        ],
    )
    def k(tc_hbm, sc_hbm, dst_hbm, out_hbm, tc_buf, sc_buf, dst_buf, acc):
        c = lax.axis_index("c")
        s = lax.axis_index("s")
        wid = s * NC + c

        @pl.loop(0, B)
        def _z(i):
            tc_buf[i, :] = jnp.zeros((F,), jnp.float32)

        pltpu.sync_copy(tc_buf, acc.at[pl.ds(s * ZR, B), :])
        pltpu.sync_copy(tc_buf.at[pl.ds(0, ZR - B), :],
                        acc.at[pl.ds(s * ZR + B, ZR - B), :])
        plsc.subcore_barrier()

        @pl.loop(0, NB)
        def _blk(blk):
            base = wid * EW + blk * B
            j = wid * NB + blk
            pltpu.sync_copy(
                tc_hbm.at[pl.ds((j // 8) * B, B), pl.ds((j % 8) * F, F)],
                tc_buf)
            pltpu.sync_copy(sc_hbm.at[pl.ds(0, 1), pl.ds(base, B)], sc_buf)
            pltpu.sync_copy(dst_hbm.at[pl.ds(1, 1), pl.ds(base, B)], dst_buf)

            @plsc.parallel_loop(0, B // 16, 1, unroll=2)
            def _mul(gi):
                g = sc_buf[0, pl.ds(gi * 16, 16)]
                for r in range(16):
                    tc_buf[gi * 16 + r, :] = tc_buf[gi * 16 + r, :] * g[r]

            for q in range(K):
                pltpu.sync_copy(tc_buf.at[pl.ds(q * SB, SB), :],
                                acc.at[dst_buf.at[0, pl.ds(q * SB, SB)]],
                                add=True)

        plsc.subcore_barrier()

        pltpu.sync_copy(acc.at[pl.ds(s * ZR, B), :], tc_buf)
        pltpu.sync_copy(tc_buf, out_hbm.at[c, pl.ds(s * ZR, B), :])
        pltpu.sync_copy(acc.at[pl.ds(s * ZR + B, ZR - B), :],
                        tc_buf.at[pl.ds(0, ZR - B), :])
        pltpu.sync_copy(tc_buf.at[pl.ds(0, ZR - B), :],
                        out_hbm.at[c, pl.ds(s * ZR + B, ZR - B), :])

    return k(tc2, sc1, dst3)


def _tc_transpose(tcT):
    Bt = 8 * B
    G = E // Bt

    def tr(x_ref, o_ref):
        row = jax.lax.broadcasted_iota(jnp.int32, (F, 8 * F), 0)
        col = jax.lax.broadcasted_iota(jnp.int32, (F, 8 * F), 1)
        acc = jnp.zeros((B, 8 * F), jnp.float32)
        for g in range(8):
            sel_g = jnp.where(col == row + g * F, 1.0, 0.0).astype(jnp.float32)
            acc = acc + jax.lax.dot_general(
                x_ref[:, g * B:(g + 1) * B], sel_g, (((0,), (0,)), ((), ())),
                preferred_element_type=jnp.float32)
        o_ref[...] = acc

    return pl.pallas_call(
        tr,
        grid=(G,),
        in_specs=[pl.BlockSpec((F, Bt), lambda i: (0, i))],
        out_specs=pl.BlockSpec((B, 8 * F), lambda i: (i, 0)),
        out_shape=jax.ShapeDtypeStruct((E // 8, 8 * F), jnp.float32),
        compiler_params=pltpu.CompilerParams(fuse_transposed_lhs_in_matmul=True),
    )(tcT)


def _tc_linear(parts3, Wt, b2):
    NR = N // 8
    NPR = NP // 8

    def mm(p_ref, w_ref, b_ref, o_ref):
        p = p_ref[0] + p_ref[1]
        for g in range(8):
            ag = p[:NR, g * F:(g + 1) * F]
            o_ref[:, g, :] = lax.dot_general(
                ag, w_ref[...], (((1,), (0,)), ((), ())),
                preferred_element_type=jnp.float32) + b_ref[...]

    return pl.pallas_call(
        mm,
        out_shape=jax.ShapeDtypeStruct((NR, 8, OUT), jnp.float32),
    )(parts3, Wt, b2)


def kernel(feature, edge_tc, edge_sc, W, b, edge_index):
    del feature
    m2 = _tc_transpose(edge_tc.T)
    parts = _sc_segment_sum(m2, edge_sc.T, edge_index)
    parts3 = parts.reshape(NC, NP // 8, 8 * F)
    out3 = _tc_linear(parts3, W.T, b.reshape(1, OUT))
    return out3.reshape(N, OUT)


# speedup vs baseline: 1.6544x; 1.1108x over previous
import functools

import jax
import jax.numpy as jnp
from jax import lax
from jax.experimental import pallas as pl
from jax.experimental.pallas import tpu as pltpu
from jax.experimental.pallas import tpu_sc as plsc

N = 50000
E = 1600000
F = 16
OUT = 128
NC, NS = 2, 16
NW = NC * NS
EW = E // NW
B = 2000
NB = EW // B
SB = 80
K = B // SB
NP = 50048
ZR = NP // NS
BP = 2048


def _sc_segment_sum(tc2, sc1, dst3):
    mesh = plsc.VectorSubcoreMesh(core_axis_name="c", subcore_axis_name="s")

    @functools.partial(
        pl.kernel,
        out_type=jax.ShapeDtypeStruct((NC, NP, F), jnp.float32),
        mesh=mesh,
        compiler_params=pltpu.CompilerParams(use_tc_tiling_on_sc=False),
        scratch_types=[
            pltpu.VMEM((B, F), jnp.float32),
            pltpu.VMEM((1, B), jnp.float32),
            pltpu.VMEM((1, B), jnp.int32),
            pltpu.VMEM_SHARED((NP, F), jnp.float32),
            pltpu.SemaphoreType.DMA,
            pltpu.SemaphoreType.DMA,
        ],
    )
    def k(tc_hbm, sc_hbm, dst_hbm, out_hbm, tc_buf, sc_buf, dst_buf, acc,
          in_sem, st_sem):
        c = lax.axis_index("c")
        s = lax.axis_index("s")
        wid = s * NC + c

        @pl.loop(0, B)
        def _z(i):
            tc_buf[i, :] = jnp.zeros((F,), jnp.float32)

        pltpu.sync_copy(tc_buf, acc.at[pl.ds(s * ZR, B), :])
        pltpu.sync_copy(tc_buf.at[pl.ds(0, ZR - B), :],
                        acc.at[pl.ds(s * ZR + B, ZR - B), :])
        plsc.subcore_barrier()

        @pl.loop(0, NB)
        def _blk(blk):
            base = wid * EW + blk * B
            j = wid * NB + blk
            h1 = pltpu.async_copy(
                tc_hbm.at[pl.ds((j // 8) * B, B), pl.ds((j % 8) * F, F)],
                tc_buf, in_sem)
            h2 = pltpu.async_copy(sc_hbm.at[pl.ds(0, 1), pl.ds(base, B)],
                                  sc_buf, in_sem)
            h3 = pltpu.async_copy(dst_hbm.at[pl.ds(1, 1), pl.ds(base, B)],
                                  dst_buf, in_sem)
            h1.wait()
            h2.wait()
            h3.wait()

            @plsc.parallel_loop(0, B // 16, 1, unroll=2)
            def _mul(gi):
                g = sc_buf[0, pl.ds(gi * 16, 16)]
                for r in range(16):
                    tc_buf[gi * 16 + r, :] = tc_buf[gi * 16 + r, :] * g[r]

            hs = [pltpu.async_copy(tc_buf.at[pl.ds(q * SB, SB), :],
                                   acc.at[dst_buf.at[0, pl.ds(q * SB, SB)]],
                                   st_sem, add=True)
                  for q in range(K)]
            for h in hs:
                h.wait()

        plsc.subcore_barrier()

        pltpu.sync_copy(acc.at[pl.ds(s * ZR, B), :], tc_buf)
        pltpu.sync_copy(tc_buf, out_hbm.at[c, pl.ds(s * ZR, B), :])
        pltpu.sync_copy(acc.at[pl.ds(s * ZR + B, ZR - B), :],
                        tc_buf.at[pl.ds(0, ZR - B), :])
        pltpu.sync_copy(tc_buf.at[pl.ds(0, ZR - B), :],
                        out_hbm.at[c, pl.ds(s * ZR + B, ZR - B), :])

    return k(tc2, sc1, dst3)


def _tc_transpose(tcT):
    Bt = 8 * B
    G = E // Bt

    def tr(x_ref, o_ref):
        row = jax.lax.broadcasted_iota(jnp.int32, (F, 8 * F), 0)
        col = jax.lax.broadcasted_iota(jnp.int32, (F, 8 * F), 1)
        acc = jnp.zeros((B, 8 * F), jnp.float32)
        for g in range(8):
            sel_g = jnp.where(col == row + g * F, 1.0, 0.0).astype(jnp.float32)
            acc = acc + jax.lax.dot_general(
                x_ref[:, g * B:(g + 1) * B], sel_g, (((0,), (0,)), ((), ())),
                preferred_element_type=jnp.float32)
        o_ref[...] = acc

    return pl.pallas_call(
        tr,
        grid=(G,),
        in_specs=[pl.BlockSpec((F, Bt), lambda i: (0, i))],
        out_specs=pl.BlockSpec((B, 8 * F), lambda i: (i, 0)),
        out_shape=jax.ShapeDtypeStruct((E // 8, 8 * F), jnp.float32),
        compiler_params=pltpu.CompilerParams(fuse_transposed_lhs_in_matmul=True),
    )(tcT)


def _tc_linear(parts3, Wt, b2):
    NR = N // 8
    NPR = NP // 8

    def mm(p_ref, w_ref, b_ref, o_ref):
        p = p_ref[0] + p_ref[1]
        for g in range(8):
            ag = p[:NR, g * F:(g + 1) * F]
            o_ref[:, g, :] = lax.dot_general(
                ag, w_ref[...], (((1,), (0,)), ((), ())),
                preferred_element_type=jnp.float32) + b_ref[...]

    return pl.pallas_call(
        mm,
        out_shape=jax.ShapeDtypeStruct((NR, 8, OUT), jnp.float32),
    )(parts3, Wt, b2)


def kernel(feature, edge_tc, edge_sc, W, b, edge_index):
    del feature
    m2 = _tc_transpose(edge_tc.T)
    parts = _sc_segment_sum(m2, edge_sc.T, edge_index)
    parts3 = parts.reshape(NC, NP // 8, 8 * F)
    out3 = _tc_linear(parts3, W.T, b.reshape(1, OUT))
    return out3.reshape(N, OUT)
